# baseline (speedup 1.0000x reference)
import functools

import jax
import jax.numpy as jnp
from jax import lax
from jax.experimental import pallas as pl
from jax.experimental.pallas import tpu as pltpu
from jax.experimental.pallas import tpu_sc as plsc

_VOCAB = 1000
_HIDDEN = 128
_BATCH = 1024
_SEQ = 50

_NC, _NS = 2, 16
_NW = _NC * _NS
_B = _BATCH * _SEQ
_BPW = _B // _NW
_CHUNK = 40
_NCHUNK = _BPW // _CHUNK


def _table_body(e_ref, wt_ref, b_ref, m_ref):
    m_ref[...] = (
        jnp.dot(e_ref[...], wt_ref[...], preferred_element_type=jnp.float32)
        + b_ref[...]
    )


def _gather_body(table_hbm, idx_hbm, out_hbm, idx_v, rows_v, sem):
    wid = lax.axis_index("s") * _NC + lax.axis_index("c")
    base = wid * _BPW
    pltpu.sync_copy(idx_hbm.at[pl.ds(base, _BPW)], idx_v)

    def chunk(c, carry):
        start = c * _CHUNK
        pltpu.async_copy(
            table_hbm.at[idx_v.at[pl.ds(start, _CHUNK)]], rows_v, sem
        ).wait()
        pltpu.sync_copy(rows_v, out_hbm.at[pl.ds(base + start, _CHUNK)])
        return carry

    lax.fori_loop(0, _NCHUNK, chunk, 0)


def kernel(input_ids, embed_table, head_w, head_b):
    table = pl.pallas_call(
        _table_body,
        out_shape=jax.ShapeDtypeStruct((_VOCAB, _VOCAB), jnp.float32),
    )(embed_table, head_w.T, head_b.reshape(1, _VOCAB))

    idx = input_ids.reshape(_B).astype(jnp.int32)
    mesh = plsc.VectorSubcoreMesh(
        core_axis_name="c", subcore_axis_name="s",
        num_cores=_NC, num_subcores=_NS,
    )
    out = pl.kernel(
        _gather_body,
        out_type=jax.ShapeDtypeStruct((_B, _VOCAB), jnp.float32),
        mesh=mesh,
        compiler_params=pltpu.CompilerParams(use_tc_tiling_on_sc=False),
        scratch_types=[
            pltpu.VMEM((_BPW,), jnp.int32),
            pltpu.VMEM((_CHUNK, _VOCAB), jnp.float32),
            pltpu.SemaphoreType.DMA,
        ],
    )(table, idx)
    return out.reshape(_BATCH, _SEQ, _VOCAB)


# speedup vs baseline: 1.0372x; 1.0372x over previous
import functools

import jax
import jax.numpy as jnp
from jax import lax
from jax.experimental import pallas as pl
from jax.experimental.pallas import tpu as pltpu
from jax.experimental.pallas import tpu_sc as plsc

_VOCAB = 1000
_HIDDEN = 128
_BATCH = 1024
_SEQ = 50

_NC, _NS = 2, 16
_NW = _NC * _NS
_B = _BATCH * _SEQ
_BPW = _B // _NW
_CHUNK = 16
_NCHUNK = _BPW // _CHUNK
_NBUF = 4


def _table_body(e_ref, wt_ref, b_ref, m_ref):
    m_ref[...] = (
        jnp.dot(e_ref[...], wt_ref[...], preferred_element_type=jnp.float32)
        + b_ref[...]
    )


def _gather_body(table_hbm, idx_hbm, out_hbm, idx_v, rows_v,
                 g0, g1, g2, g3, w0, w1, w2, w3):
    gsems = (g0, g1, g2, g3)
    wsems = (w0, w1, w2, w3)
    wid = lax.axis_index("s") * _NC + lax.axis_index("c")
    base = wid * _BPW
    pltpu.sync_copy(idx_hbm.at[pl.ds(base, _BPW)], idx_v)

    def start_gather(k, b):
        pltpu.async_copy(
            table_hbm.at[idx_v.at[pl.ds(k * _CHUNK, _CHUNK)]],
            rows_v.at[b], gsems[b])

    def wait_gather(b):
        pltpu.make_async_copy(
            table_hbm.at[pl.ds(0, _CHUNK)], rows_v.at[b], gsems[b]).wait()

    def start_write(k, b):
        pltpu.async_copy(
            rows_v.at[b], out_hbm.at[pl.ds(base + k * _CHUNK, _CHUNK)],
            wsems[b])

    def wait_write(b):
        pltpu.make_async_copy(
            rows_v.at[b], out_hbm.at[pl.ds(base, _CHUNK)], wsems[b]).wait()

    def slot(k, b, head, tail):
        if not tail:
            bb = (b + 2) % _NBUF
            if not head:
                wait_write(bb)
            start_gather(k + 2, bb)
        wait_gather(b)
        start_write(k, b)

    start_gather(0, 0)
    start_gather(1, 1)
    for b in range(_NBUF):
        slot(b, b, head=(b < 2), tail=False)

    def group(c, carry):
        k0 = c * _NBUF
        for b in range(_NBUF):
            slot(k0 + b, b, head=False, tail=False)
        return carry

    lax.fori_loop(1, _NCHUNK // _NBUF - 1, group, 0)

    kl = _NCHUNK - _NBUF
    for b in range(_NBUF):
        slot(kl + b, b, head=False, tail=(b >= 2))
    for b in range(_NBUF):
        wait_write(b)


def kernel(input_ids, embed_table, head_w, head_b):
    table = pl.pallas_call(
        _table_body,
        out_shape=jax.ShapeDtypeStruct((_VOCAB, _VOCAB), jnp.float32),
    )(embed_table, head_w.T, head_b.reshape(1, _VOCAB))

    idx = input_ids.reshape(_B).astype(jnp.int32)
    mesh = plsc.VectorSubcoreMesh(
        core_axis_name="c", subcore_axis_name="s",
        num_cores=_NC, num_subcores=_NS,
    )
    out = pl.kernel(
        _gather_body,
        out_type=jax.ShapeDtypeStruct((_B, _VOCAB), jnp.float32),
        mesh=mesh,
        compiler_params=pltpu.CompilerParams(use_tc_tiling_on_sc=False),
        scratch_types=[
            pltpu.VMEM((_BPW,), jnp.int32),
            pltpu.VMEM((_NBUF, _CHUNK, _VOCAB), jnp.float32),
        ] + [pltpu.SemaphoreType.DMA] * (2 * _NBUF),
    )(table, idx)
    return out.reshape(_BATCH, _SEQ, _VOCAB)
